# baseline (speedup 1.0000x reference)
import jax
import jax.numpy as jnp
from jax import lax
from jax.experimental import pallas as pl
from jax.experimental.pallas import tpu as pltpu
from jax.experimental.pallas import tpu_sc as plsc

S = 4096
D = 768
NE = 8
BLK = 512
NT = S // BLK
EB = 128
P = S + NE * EB
NBLK = P // EB
NBLK_PAD = 48
NC = 2
NS = 16
NW = NC * NS
CHUNK = S // NW
L = 16


def _stage1_body(x_ref, encw_ref, encb_ref, gatew_ref, coefw_ref, coefb_ref,
                 mlpw_ref, mlpb_ref, expb_ref,
                 hs_ref, base_ref, idx_ref):
    h = jnp.maximum(x_ref[...] @ encw_ref[...] + encb_ref[...], 0.0)
    logits = h @ gatew_ref[...]
    m = jnp.max(logits, axis=-1, keepdims=True)
    ssum = jnp.sum(jnp.exp(logits - m), axis=-1, keepdims=True)
    top_gate = 1.0 / ssum
    lane = lax.broadcasted_iota(jnp.int32, logits.shape, 1)
    idx = jnp.min(jnp.where(logits == m, lane, NE), axis=-1, keepdims=True)
    z = h @ coefw_ref[...] + coefb_ref[...]
    zm = jnp.max(z, axis=-1, keepdims=True)
    ze = jnp.exp(z - zm)
    c = ze / jnp.sum(ze, axis=-1, keepdims=True)
    scale = top_gate * c[:, 0:1]
    hs_ref[...] = h * scale
    onehot = jnp.where(lane == idx, 1.0, 0.0)
    base_ref[...] = ((h @ mlpw_ref[...] + mlpb_ref[...]) * c[:, 1:2]
                     + (onehot @ expb_ref[...]) * scale)
    idx_ref[...] = idx


def _stage1(x, enc_W, enc_b, gate_W, coef_W, coef_b, mlp_W, mlp_b, expert_b):
    full = lambda r, c: pl.BlockSpec((r, c), lambda i: (0, 0))
    return pl.pallas_call(
        _stage1_body,
        grid=(NT,),
        in_specs=[
            pl.BlockSpec((BLK, D), lambda i: (i, 0)),
            full(D, D), full(1, D), full(D, NE), full(D, 2), full(1, 2),
            full(D, D), full(1, D), full(NE, D),
        ],
        out_specs=[
            pl.BlockSpec((BLK, D), lambda i: (i, 0)),
            pl.BlockSpec((BLK, D), lambda i: (i, 0)),
            pl.BlockSpec((BLK, 1), lambda i: (i, 0)),
        ],
        out_shape=[
            jax.ShapeDtypeStruct((S, D), jnp.float32),
            jax.ShapeDtypeStruct((S, D), jnp.float32),
            jax.ShapeDtypeStruct((S, 1), jnp.int32),
        ],
    )(x, enc_W, enc_b.reshape(1, D), gate_W, coef_W, coef_b.reshape(1, 2),
      mlp_W, mlp_b.reshape(1, D), expert_b)


def _router_body(idx_hbm, hs_hbm, hsort_hbm, pos_hbm, b2e_hbm,
                 idx_all, pos_v, rows_v, b2e_v, sem):
    wid = lax.axis_index("s") * NC + lax.axis_index("c")
    base = wid * CHUNK
    pltpu.sync_copy(idx_hbm, idx_all)

    zero = jnp.int32(0)

    def hist_step(i, carry):
        tot, pre = carry
        v = idx_all[pl.ds(i * L, L)]
        before = i * L < base
        new_tot = []
        new_pre = []
        for e in range(NE):
            cnt = jnp.sum(jnp.where(v == e, 1, 0))
            new_tot.append(tot[e] + cnt)
            new_pre.append(pre[e] + jnp.where(before, cnt, zero))
        return tuple(new_tot), tuple(new_pre)

    tot, pre = lax.fori_loop(0, S // L, hist_step,
                             ((zero,) * NE, (zero,) * NE))

    off = [zero]
    for e in range(NE):
        padc = ((tot[e] + (EB - 1)) // EB) * EB
        off.append(off[-1] + padc)

    start = [off[e] + pre[e] for e in range(NE)]
    for k in range(CHUNK // L):
        v = idx_all[pl.ds(base + k * L, L)]
        pos = jnp.zeros((L,), jnp.int32)
        for e in range(NE):
            msk = v == e
            mi = jnp.where(msk, 1, 0)
            cs = plsc.cumsum(mi)
            pos = jnp.where(msk, start[e] + cs - 1, pos)
            start[e] = start[e] + jnp.sum(mi)
        pos_v[pl.ds(k * L, L)] = pos

    pltpu.sync_copy(pos_v, pos_hbm.at[pl.ds(base, CHUNK)])
    pltpu.sync_copy(hs_hbm.at[pl.ds(base, CHUNK)], rows_v)
    pltpu.async_copy(rows_v, hsort_hbm.at[pos_v], sem).wait()

    @pl.when(wid == 0)
    def _b2e():
        for t in range(NBLK_PAD // L):
            bstart = (lax.iota(jnp.int32, L) + t * L) * EB
            acc = zero
            for e in range(1, NE):
                acc = acc + jnp.where(bstart >= off[e], 1, 0)
            b2e_v[pl.ds(t * L, L)] = acc
        pltpu.sync_copy(b2e_v, b2e_hbm)


def _router(idx, hs):
    mesh = plsc.VectorSubcoreMesh(core_axis_name="c", subcore_axis_name="s")
    return pl.kernel(
        _router_body,
        out_type=[
            jax.ShapeDtypeStruct((P, D), jnp.float32),
            jax.ShapeDtypeStruct((S,), jnp.int32),
            jax.ShapeDtypeStruct((NBLK_PAD,), jnp.int32),
        ],
        mesh=mesh,
        compiler_params=pltpu.CompilerParams(needs_layout_passes=False),
        scratch_types=[
            pltpu.VMEM((S,), jnp.int32),
            pltpu.VMEM((CHUNK,), jnp.int32),
            pltpu.VMEM((CHUNK, D), jnp.float32),
            pltpu.VMEM((NBLK_PAD,), jnp.int32),
            pltpu.SemaphoreType.DMA,
        ],
    )(idx, hs)


def _expert_body(b2e_ref, hs_ref, ew_ref, out_ref):
    out_ref[...] = jnp.dot(hs_ref[...], ew_ref[0],
                           preferred_element_type=jnp.float32)


def _expert_mm(b2e, hsort, expert_W):
    grid_spec = pltpu.PrefetchScalarGridSpec(
        num_scalar_prefetch=1,
        grid=(NBLK,),
        in_specs=[
            pl.BlockSpec((EB, D), lambda i, b2e: (i, 0)),
            pl.BlockSpec((1, D, D), lambda i, b2e: (b2e[i], 0, 0)),
        ],
        out_specs=pl.BlockSpec((EB, D), lambda i, b2e: (i, 0)),
    )
    return pl.pallas_call(
        _expert_body,
        grid_spec=grid_spec,
        out_shape=jax.ShapeDtypeStruct((P, D), jnp.float32),
    )(b2e, hsort, expert_W)


def _gather_body(ye_hbm, pos_hbm, moe_hbm, pos_v, rows_v, sem):
    wid = lax.axis_index("s") * NC + lax.axis_index("c")
    base = wid * CHUNK
    pltpu.sync_copy(pos_hbm.at[pl.ds(base, CHUNK)], pos_v)
    pltpu.async_copy(ye_hbm.at[pos_v], rows_v, sem).wait()
    pltpu.sync_copy(rows_v, moe_hbm.at[pl.ds(base, CHUNK)])


def _gather(ye, pos):
    mesh = plsc.VectorSubcoreMesh(core_axis_name="c", subcore_axis_name="s")
    return pl.kernel(
        _gather_body,
        out_type=jax.ShapeDtypeStruct((S, D), jnp.float32),
        mesh=mesh,
        compiler_params=pltpu.CompilerParams(needs_layout_passes=False),
        scratch_types=[
            pltpu.VMEM((CHUNK,), jnp.int32),
            pltpu.VMEM((CHUNK, D), jnp.float32),
            pltpu.SemaphoreType.DMA,
        ],
    )(ye, pos)


def _final_body(moe_ref, base_ref, decw_ref, decb_ref, out_ref):
    out_ref[...] = ((moe_ref[...] + base_ref[...]) @ decw_ref[...]
                    + decb_ref[...])


def _final(moe, base, dec_W, dec_b):
    full = lambda r, c: pl.BlockSpec((r, c), lambda i: (0, 0))
    return pl.pallas_call(
        _final_body,
        grid=(NT,),
        in_specs=[
            pl.BlockSpec((BLK, D), lambda i: (i, 0)),
            pl.BlockSpec((BLK, D), lambda i: (i, 0)),
            full(D, D), full(1, D),
        ],
        out_specs=pl.BlockSpec((BLK, D), lambda i: (i, 0)),
        out_shape=jax.ShapeDtypeStruct((S, D), jnp.float32),
    )(moe, base, dec_W, dec_b.reshape(1, D))


def kernel(x, enc_W, enc_b, gate_W, expert_W, expert_b, mlp_W, mlp_b,
           coef_W, coef_b, dec_W, dec_b):
    hs, base, idx = _stage1(x, enc_W, enc_b, gate_W, coef_W, coef_b,
                            mlp_W, mlp_b, expert_b)
    hsort, pos, b2e = _router(idx.reshape(S), hs)
    ye = _expert_mm(b2e, hsort, expert_W)
    moe = _gather(ye, pos)
    return _final(moe, base, dec_W, dec_b)


# speedup vs baseline: 1.0044x; 1.0044x over previous
import jax
import jax.numpy as jnp
from jax import lax
from jax.experimental import pallas as pl
from jax.experimental.pallas import tpu as pltpu
from jax.experimental.pallas import tpu_sc as plsc

S = 4096
D = 768
NE = 8
BLK = 512
NT = S // BLK
EB = 128
P = S + NE * EB
NBLK = P // EB
NBLK_PAD = 48
NC = 2
NS = 16
NW = NC * NS
CHUNK = S // NW
L = 16


def _stage1_body(x_ref, encw_ref, encb_ref, gatew_ref, coefw_ref, coefb_ref,
                 mlpw_ref, mlpb_ref, expb_ref,
                 hs_ref, base_ref, idx_ref):
    h = jnp.maximum(x_ref[...] @ encw_ref[...] + encb_ref[...], 0.0)
    logits = h @ gatew_ref[...]
    m = jnp.max(logits, axis=-1, keepdims=True)
    ssum = jnp.sum(jnp.exp(logits - m), axis=-1, keepdims=True)
    top_gate = 1.0 / ssum
    lane = lax.broadcasted_iota(jnp.int32, logits.shape, 1)
    idx = jnp.min(jnp.where(logits == m, lane, NE), axis=-1, keepdims=True)
    z = h @ coefw_ref[...] + coefb_ref[...]
    zm = jnp.max(z, axis=-1, keepdims=True)
    ze = jnp.exp(z - zm)
    c = ze / jnp.sum(ze, axis=-1, keepdims=True)
    scale = top_gate * c[:, 0:1]
    hs_ref[...] = h * scale
    onehot = jnp.where(lane == idx, 1.0, 0.0)
    base_ref[...] = ((h @ mlpw_ref[...] + mlpb_ref[...]) * c[:, 1:2]
                     + (onehot @ expb_ref[...]) * scale)
    idx_ref[...] = idx


def _stage1(x, enc_W, enc_b, gate_W, coef_W, coef_b, mlp_W, mlp_b, expert_b):
    full = lambda r, c: pl.BlockSpec((r, c), lambda i: (0, 0))
    return pl.pallas_call(
        _stage1_body,
        grid=(NT,),
        in_specs=[
            pl.BlockSpec((BLK, D), lambda i: (i, 0)),
            full(D, D), full(1, D), full(D, NE), full(D, 2), full(1, 2),
            full(D, D), full(1, D), full(NE, D),
        ],
        out_specs=[
            pl.BlockSpec((BLK, D), lambda i: (i, 0)),
            pl.BlockSpec((BLK, D), lambda i: (i, 0)),
            pl.BlockSpec((BLK, 1), lambda i: (i, 0)),
        ],
        out_shape=[
            jax.ShapeDtypeStruct((S, D), jnp.float32),
            jax.ShapeDtypeStruct((S, D), jnp.float32),
            jax.ShapeDtypeStruct((S, 1), jnp.int32),
        ],
    )(x, enc_W, enc_b.reshape(1, D), gate_W, coef_W, coef_b.reshape(1, 2),
      mlp_W, mlp_b.reshape(1, D), expert_b)


def _router_body(idx_hbm, hs_hbm, hsort_hbm, pos_hbm, b2e_hbm,
                 idx_all, pos_v, rows_v, b2e_v, sem):
    wid = lax.axis_index("s") * NC + lax.axis_index("c")
    base = wid * CHUNK
    pltpu.sync_copy(idx_hbm, idx_all)

    zero = jnp.int32(0)

    def hist_step(i, carry):
        tot, pre = carry
        v = idx_all[pl.ds(i * L, L)]
        before = i * L < base
        new_tot = []
        new_pre = []
        for e in range(NE):
            cnt = jnp.sum(jnp.where(v == e, 1, 0))
            new_tot.append(tot[e] + cnt)
            new_pre.append(pre[e] + jnp.where(before, cnt, zero))
        return tuple(new_tot), tuple(new_pre)

    tot, pre = lax.fori_loop(0, S // L, hist_step,
                             ((zero,) * NE, (zero,) * NE))

    off = [zero]
    for e in range(NE):
        padc = ((tot[e] + (EB - 1)) // EB) * EB
        off.append(off[-1] + padc)

    start = [off[e] + pre[e] for e in range(NE)]
    for k in range(CHUNK // L):
        v = idx_all[pl.ds(base + k * L, L)]
        pos = jnp.zeros((L,), jnp.int32)
        for e in range(NE):
            msk = v == e
            mi = jnp.where(msk, 1, 0)
            cs = plsc.cumsum(mi)
            pos = jnp.where(msk, start[e] + cs - 1, pos)
            start[e] = start[e] + jnp.sum(mi)
        pos_v[pl.ds(k * L, L)] = pos

    pltpu.sync_copy(pos_v, pos_hbm.at[pl.ds(base, CHUNK)])
    pltpu.sync_copy(hs_hbm.at[pl.ds(base, CHUNK)], rows_v)
    pltpu.async_copy(rows_v, hsort_hbm.at[pos_v], sem).wait()

    @pl.when(wid == 0)
    def _b2e():
        for t in range(NBLK_PAD // L):
            bstart = (lax.iota(jnp.int32, L) + t * L) * EB
            acc = zero
            for e in range(1, NE):
                acc = acc + jnp.where(bstart >= off[e], 1, 0)
            b2e_v[pl.ds(t * L, L)] = acc
        pltpu.sync_copy(b2e_v, b2e_hbm)


def _router(idx, hs):
    mesh = plsc.VectorSubcoreMesh(core_axis_name="c", subcore_axis_name="s")
    return pl.kernel(
        _router_body,
        out_type=[
            jax.ShapeDtypeStruct((P, D), jnp.float32),
            jax.ShapeDtypeStruct((S,), jnp.int32),
            jax.ShapeDtypeStruct((NBLK_PAD,), jnp.int32),
        ],
        mesh=mesh,
        compiler_params=pltpu.CompilerParams(needs_layout_passes=False),
        scratch_types=[
            pltpu.VMEM((S,), jnp.int32),
            pltpu.VMEM((CHUNK,), jnp.int32),
            pltpu.VMEM((CHUNK, D), jnp.float32),
            pltpu.VMEM((NBLK_PAD,), jnp.int32),
            pltpu.SemaphoreType.DMA,
        ],
    )(idx, hs)


def _expert_body(b2e_ref, hs_ref, ew_ref, out_ref):
    e = b2e_ref[pl.program_id(0)]
    out_ref[...] = jnp.dot(hs_ref[...], ew_ref[e],
                           preferred_element_type=jnp.float32)


def _expert_mm(b2e, hsort, expert_W):
    grid_spec = pltpu.PrefetchScalarGridSpec(
        num_scalar_prefetch=1,
        grid=(NBLK,),
        in_specs=[
            pl.BlockSpec((EB, D), lambda i, b2e: (i, 0)),
            pl.BlockSpec((NE, D, D), lambda i, b2e: (0, 0, 0)),
        ],
        out_specs=pl.BlockSpec((EB, D), lambda i, b2e: (i, 0)),
    )
    return pl.pallas_call(
        _expert_body,
        grid_spec=grid_spec,
        out_shape=jax.ShapeDtypeStruct((P, D), jnp.float32),
    )(b2e, hsort, expert_W)


def _gather_body(ye_hbm, pos_hbm, moe_hbm, pos_v, rows_v, sem):
    wid = lax.axis_index("s") * NC + lax.axis_index("c")
    base = wid * CHUNK
    pltpu.sync_copy(pos_hbm.at[pl.ds(base, CHUNK)], pos_v)
    pltpu.async_copy(ye_hbm.at[pos_v], rows_v, sem).wait()
    pltpu.sync_copy(rows_v, moe_hbm.at[pl.ds(base, CHUNK)])


def _gather(ye, pos):
    mesh = plsc.VectorSubcoreMesh(core_axis_name="c", subcore_axis_name="s")
    return pl.kernel(
        _gather_body,
        out_type=jax.ShapeDtypeStruct((S, D), jnp.float32),
        mesh=mesh,
        compiler_params=pltpu.CompilerParams(needs_layout_passes=False),
        scratch_types=[
            pltpu.VMEM((CHUNK,), jnp.int32),
            pltpu.VMEM((CHUNK, D), jnp.float32),
            pltpu.SemaphoreType.DMA,
        ],
    )(ye, pos)


def _final_body(moe_ref, base_ref, decw_ref, decb_ref, out_ref):
    out_ref[...] = ((moe_ref[...] + base_ref[...]) @ decw_ref[...]
                    + decb_ref[...])


def _final(moe, base, dec_W, dec_b):
    full = lambda r, c: pl.BlockSpec((r, c), lambda i: (0, 0))
    return pl.pallas_call(
        _final_body,
        grid=(NT,),
        in_specs=[
            pl.BlockSpec((BLK, D), lambda i: (i, 0)),
            pl.BlockSpec((BLK, D), lambda i: (i, 0)),
            full(D, D), full(1, D),
        ],
        out_specs=pl.BlockSpec((BLK, D), lambda i: (i, 0)),
        out_shape=jax.ShapeDtypeStruct((S, D), jnp.float32),
    )(moe, base, dec_W, dec_b.reshape(1, D))


def kernel(x, enc_W, enc_b, gate_W, expert_W, expert_b, mlp_W, mlp_b,
           coef_W, coef_b, dec_W, dec_b):
    hs, base, idx = _stage1(x, enc_W, enc_b, gate_W, coef_W, coef_b,
                            mlp_W, mlp_b, expert_b)
    hsort, pos, b2e = _router(idx.reshape(S), hs)
    ye = _expert_mm(b2e, hsort, expert_W)
    moe = _gather(ye, pos)
    return _final(moe, base, dec_W, dec_b)


# speedup vs baseline: 1.0612x; 1.0566x over previous
import jax
import jax.numpy as jnp
from jax import lax
from jax.experimental import pallas as pl
from jax.experimental.pallas import tpu as pltpu
from jax.experimental.pallas import tpu_sc as plsc

S = 4096
D = 768
NE = 8
BLK = 512
NT = S // BLK
EB = 256
P = S + NE * EB
NBLK = P // EB
NBLK_PAD = 32
NC = 2
NS = 16
NW = NC * NS
CHUNK = S // NW
L = 16


def _stage1_body(x_ref, encw_ref, encb_ref, gatew_ref, coefw_ref, coefb_ref,
                 mlpw_ref, mlpb_ref, expb_ref,
                 hs_ref, base_ref, idx_ref):
    h = jnp.maximum(x_ref[...] @ encw_ref[...] + encb_ref[...], 0.0)
    logits = h @ gatew_ref[...]
    m = jnp.max(logits, axis=-1, keepdims=True)
    ssum = jnp.sum(jnp.exp(logits - m), axis=-1, keepdims=True)
    top_gate = 1.0 / ssum
    lane = lax.broadcasted_iota(jnp.int32, logits.shape, 1)
    idx = jnp.min(jnp.where(logits == m, lane, NE), axis=-1, keepdims=True)
    z = h @ coefw_ref[...] + coefb_ref[...]
    zm = jnp.max(z, axis=-1, keepdims=True)
    ze = jnp.exp(z - zm)
    c = ze / jnp.sum(ze, axis=-1, keepdims=True)
    scale = top_gate * c[:, 0:1]
    hs_ref[...] = h * scale
    onehot = jnp.where(lane == idx, 1.0, 0.0)
    base_ref[...] = ((h @ mlpw_ref[...] + mlpb_ref[...]) * c[:, 1:2]
                     + (onehot @ expb_ref[...]) * scale)
    idx_ref[...] = idx


def _stage1(x, enc_W, enc_b, gate_W, coef_W, coef_b, mlp_W, mlp_b, expert_b):
    full = lambda r, c: pl.BlockSpec((r, c), lambda i: (0, 0))
    return pl.pallas_call(
        _stage1_body,
        grid=(NT,),
        in_specs=[
            pl.BlockSpec((BLK, D), lambda i: (i, 0)),
            full(D, D), full(1, D), full(D, NE), full(D, 2), full(1, 2),
            full(D, D), full(1, D), full(NE, D),
        ],
        out_specs=[
            pl.BlockSpec((BLK, D), lambda i: (i, 0)),
            pl.BlockSpec((BLK, D), lambda i: (i, 0)),
            pl.BlockSpec((BLK, 1), lambda i: (i, 0)),
        ],
        out_shape=[
            jax.ShapeDtypeStruct((S, D), jnp.float32),
            jax.ShapeDtypeStruct((S, D), jnp.float32),
            jax.ShapeDtypeStruct((S, 1), jnp.int32),
        ],
    )(x, enc_W, enc_b.reshape(1, D), gate_W, coef_W, coef_b.reshape(1, 2),
      mlp_W, mlp_b.reshape(1, D), expert_b)


def _router_body(idx_hbm, hs_hbm, hsort_hbm, pos_hbm, b2e_hbm,
                 idx_all, pos_v, rows_v, b2e_v, sem):
    wid = lax.axis_index("s") * NC + lax.axis_index("c")
    base = wid * CHUNK
    pltpu.sync_copy(idx_hbm, idx_all)

    zero = jnp.int32(0)

    def hist_step(i, carry):
        tot, pre = carry
        v = idx_all[pl.ds(i * L, L)]
        before = i * L < base
        new_tot = []
        new_pre = []
        for e in range(NE):
            cnt = jnp.sum(jnp.where(v == e, 1, 0))
            new_tot.append(tot[e] + cnt)
            new_pre.append(pre[e] + jnp.where(before, cnt, zero))
        return tuple(new_tot), tuple(new_pre)

    tot, pre = lax.fori_loop(0, S // L, hist_step,
                             ((zero,) * NE, (zero,) * NE))

    off = [zero]
    for e in range(NE):
        padc = ((tot[e] + (EB - 1)) // EB) * EB
        off.append(off[-1] + padc)

    start = [off[e] + pre[e] for e in range(NE)]
    for k in range(CHUNK // L):
        v = idx_all[pl.ds(base + k * L, L)]
        pos = jnp.zeros((L,), jnp.int32)
        for e in range(NE):
            msk = v == e
            mi = jnp.where(msk, 1, 0)
            cs = plsc.cumsum(mi)
            pos = jnp.where(msk, start[e] + cs - 1, pos)
            start[e] = start[e] + jnp.sum(mi)
        pos_v[pl.ds(k * L, L)] = pos

    pltpu.sync_copy(pos_v, pos_hbm.at[pl.ds(base, CHUNK)])
    pltpu.sync_copy(hs_hbm.at[pl.ds(base, CHUNK)], rows_v)
    pltpu.async_copy(rows_v, hsort_hbm.at[pos_v], sem).wait()

    @pl.when(wid == 0)
    def _b2e():
        for t in range(NBLK_PAD // L):
            bstart = (lax.iota(jnp.int32, L) + t * L) * EB
            acc = zero
            for e in range(1, NE):
                acc = acc + jnp.where(bstart >= off[e], 1, 0)
            b2e_v[pl.ds(t * L, L)] = acc
        pltpu.sync_copy(b2e_v, b2e_hbm)


def _router(idx, hs):
    mesh = plsc.VectorSubcoreMesh(core_axis_name="c", subcore_axis_name="s")
    return pl.kernel(
        _router_body,
        out_type=[
            jax.ShapeDtypeStruct((P, D), jnp.float32),
            jax.ShapeDtypeStruct((S,), jnp.int32),
            jax.ShapeDtypeStruct((NBLK_PAD,), jnp.int32),
        ],
        mesh=mesh,
        compiler_params=pltpu.CompilerParams(needs_layout_passes=False),
        scratch_types=[
            pltpu.VMEM((S,), jnp.int32),
            pltpu.VMEM((CHUNK,), jnp.int32),
            pltpu.VMEM((CHUNK, D), jnp.float32),
            pltpu.VMEM((NBLK_PAD,), jnp.int32),
            pltpu.SemaphoreType.DMA,
        ],
    )(idx, hs)


def _expert_body(b2e_ref, hs_ref, ew_ref, out_ref):
    e = b2e_ref[pl.program_id(0)]
    out_ref[...] = jnp.dot(hs_ref[...], ew_ref[e],
                           preferred_element_type=jnp.float32)


def _expert_mm(b2e, hsort, expert_W):
    grid_spec = pltpu.PrefetchScalarGridSpec(
        num_scalar_prefetch=1,
        grid=(NBLK,),
        in_specs=[
            pl.BlockSpec((EB, D), lambda i, b2e: (i, 0)),
            pl.BlockSpec((NE, D, D), lambda i, b2e: (0, 0, 0)),
        ],
        out_specs=pl.BlockSpec((EB, D), lambda i, b2e: (i, 0)),
    )
    return pl.pallas_call(
        _expert_body,
        grid_spec=grid_spec,
        out_shape=jax.ShapeDtypeStruct((P, D), jnp.float32),
    )(b2e, hsort, expert_W)


def _gather_body(ye_hbm, pos_hbm, moe_hbm, pos_v, rows_v, sem):
    wid = lax.axis_index("s") * NC + lax.axis_index("c")
    base = wid * CHUNK
    pltpu.sync_copy(pos_hbm.at[pl.ds(base, CHUNK)], pos_v)
    pltpu.async_copy(ye_hbm.at[pos_v], rows_v, sem).wait()
    pltpu.sync_copy(rows_v, moe_hbm.at[pl.ds(base, CHUNK)])


def _gather(ye, pos):
    mesh = plsc.VectorSubcoreMesh(core_axis_name="c", subcore_axis_name="s")
    return pl.kernel(
        _gather_body,
        out_type=jax.ShapeDtypeStruct((S, D), jnp.float32),
        mesh=mesh,
        compiler_params=pltpu.CompilerParams(needs_layout_passes=False),
        scratch_types=[
            pltpu.VMEM((CHUNK,), jnp.int32),
            pltpu.VMEM((CHUNK, D), jnp.float32),
            pltpu.SemaphoreType.DMA,
        ],
    )(ye, pos)


def _final_body(moe_ref, base_ref, decw_ref, decb_ref, out_ref):
    out_ref[...] = ((moe_ref[...] + base_ref[...]) @ decw_ref[...]
                    + decb_ref[...])


def _final(moe, base, dec_W, dec_b):
    full = lambda r, c: pl.BlockSpec((r, c), lambda i: (0, 0))
    return pl.pallas_call(
        _final_body,
        grid=(NT,),
        in_specs=[
            pl.BlockSpec((BLK, D), lambda i: (i, 0)),
            pl.BlockSpec((BLK, D), lambda i: (i, 0)),
            full(D, D), full(1, D),
        ],
        out_specs=pl.BlockSpec((BLK, D), lambda i: (i, 0)),
        out_shape=jax.ShapeDtypeStruct((S, D), jnp.float32),
    )(moe, base, dec_W, dec_b.reshape(1, D))


def kernel(x, enc_W, enc_b, gate_W, expert_W, expert_b, mlp_W, mlp_b,
           coef_W, coef_b, dec_W, dec_b):
    hs, base, idx = _stage1(x, enc_W, enc_b, gate_W, coef_W, coef_b,
                            mlp_W, mlp_b, expert_b)
    hsort, pos, b2e = _router(idx.reshape(S), hs)
    ye = _expert_mm(b2e, hsort, expert_W)
    moe = _gather(ye, pos)
    return _final(moe, base, dec_W, dec_b)
